# Initial kernel scaffold; baseline (speedup 1.0000x reference)
import jax
import jax.numpy as jnp
from jax.experimental import pallas as pl


def kernel(x, edge_index, W, b):
    raise NotImplementedError("write your pallas kernel here")


# speedup vs baseline: 13.3197x; 13.3197x over previous
import functools

import jax
import jax.numpy as jnp
from jax import lax
from jax.experimental import pallas as pl
from jax.experimental.pallas import tpu as pltpu
from jax.experimental.pallas import tpu_sc as plsc

NC = 2
NS = 16
NW = NC * NS
LANES = 16

CHUNK = 128
SC_CH = 8
NBLK = 128


def _sc_mesh():
    return plsc.VectorSubcoreMesh(core_axis_name="c", subcore_axis_name="s")


def _deg_kernel_body(n_acc, ept, dst_hbm, hists_hbm, dstv, hist):
    c = lax.axis_index("c")
    s = lax.axis_index("s")
    wid = s * NC + c
    pltpu.sync_copy(dst_hbm.at[wid], dstv)

    zeros16 = jnp.zeros((LANES,), jnp.float32)

    def zero_body(i, carry):
        hist[pl.ds(i * LANES, LANES)] = zeros16
        return carry

    lax.fori_loop(0, n_acc // LANES, zero_body, 0)

    ones16 = jnp.ones((LANES,), jnp.float32)

    def body(i, carry):
        idx = dstv[pl.ds(i * LANES, LANES)]
        plsc.addupdate_scatter(hist, [idx], ones16)
        return carry

    lax.fori_loop(0, ept // LANES, body, 0)
    pltpu.sync_copy(hist, hists_hbm.at[wid])


def _g_kernel_body(x_ref, w_ref, hists_ref, g_ref):
    deg = jnp.sum(hists_ref[...], axis=0) + 1.0
    dinv = lax.rsqrt(deg)
    h = jnp.dot(x_ref[...], w_ref[...], preferred_element_type=jnp.float32)
    g_ref[...] = h * dinv[:, None]


def _edge_kernel_body(n_acc, c_t, src_hbm, dst_hbm, g_hbm, part_hbm,
                      srcv, dstv, rows, acc, sg0, sg1, si):
    cid = lax.axis_index("c")
    s = lax.axis_index("s")
    wid = s * NC + cid
    d = g_hbm.shape[1]
    n_super = c_t // SC_CH

    zeros16 = jnp.zeros((LANES,), jnp.float32)
    per_row = d // LANES

    def zb(i, carry):
        r = i // per_row
        j = i % per_row
        rows[0, r, pl.ds(j * LANES, LANES)] = zeros16
        return carry

    lax.fori_loop(0, CHUNK * per_row, zb, 0)

    r_t = n_acc // NS

    def zacc(j, carry):
        pltpu.sync_copy(rows.at[0], acc.at[pl.ds(s * r_t + j * CHUNK, CHUNK)])
        return carry

    lax.fori_loop(0, r_t // CHUNK, zacc, 0)
    plsc.subcore_barrier()

    def stage(sup):
        slot = sup % 2
        pltpu.async_copy(src_hbm.at[wid, pl.ds(sup * SC_CH, SC_CH)],
                         srcv.at[slot], si)
        pltpu.async_copy(dst_hbm.at[wid, pl.ds(sup * SC_CH, SC_CH)],
                         dstv.at[slot], si)

    def stage_wait():
        pltpu.make_async_copy(src_hbm.at[wid, pl.ds(0, SC_CH)], srcv.at[0],
                              si).wait()
        pltpu.make_async_copy(dst_hbm.at[wid, pl.ds(0, SC_CH)], dstv.at[0],
                              si).wait()

    stage(0)
    stage_wait()
    sems = (sg0, sg1)
    for b in range(2):
        pltpu.async_copy(g_hbm.at[srcv.at[0, b]], rows.at[b], sems[b])

    def pair(c2, carry):
        for b in range(2):
            cc = c2 * 2 + b
            sup = cc // SC_CH
            pltpu.make_async_copy(
                g_hbm.at[srcv.at[sup % 2, cc % SC_CH]], rows.at[b],
                sems[b]).wait()
            pltpu.sync_copy(rows.at[b], acc.at[dstv.at[sup % 2, cc % SC_CH]],
                            add=True)
            ahead = cc + 2

            @pl.when(ahead < c_t)
            def _():
                asup = ahead // SC_CH

                @pl.when(ahead % SC_CH == 0)
                def _():
                    stage_wait()

                @pl.when(jnp.logical_and(ahead % SC_CH == 2,
                                         asup + 1 < n_super))
                def _():
                    stage(asup + 1)

                pltpu.async_copy(
                    g_hbm.at[srcv.at[asup % 2, ahead % SC_CH]], rows.at[b],
                    sems[b])
        return carry

    lax.fori_loop(0, c_t // 2, pair, 0)
    plsc.subcore_barrier()

    pltpu.sync_copy(acc.at[pl.ds(s * r_t, r_t)],
                    part_hbm.at[cid, pl.ds(s * r_t, r_t)])


def _out_kernel_body(p_ref, g_ref, hists_ref, b_ref, o_ref):
    deg = jnp.sum(hists_ref[...], axis=0) + 1.0
    dinv = lax.rsqrt(deg)
    tot = p_ref[0] + p_ref[1] + g_ref[...]
    o_ref[...] = tot * dinv[:, None] + b_ref[...]


def kernel(x, edge_index, W, b):
    n, d = x.shape
    e = edge_index.shape[1]

    c_t = -(-e // (NW * CHUNK))
    c_t = -(-c_t // SC_CH) * SC_CH
    ept = c_t * CHUNK
    e_pad = NW * ept

    n_acc = -(-(n + 1) // (NS * NBLK)) * (NS * NBLK)

    src = edge_index[0].astype(jnp.int32)
    dst = edge_index[1].astype(jnp.int32)
    pad = e_pad - e
    src_p = jnp.concatenate([src, jnp.zeros((pad,), jnp.int32)])
    dst_p = jnp.concatenate([dst, jnp.full((pad,), n, jnp.int32)])
    src_r = src_p.reshape(NW, c_t, CHUNK)
    dst_r = dst_p.reshape(NW, c_t, CHUNK)
    dst_flat = dst_p.reshape(NW, ept)

    mesh = _sc_mesh()

    x_pad = jnp.concatenate([x, jnp.zeros((n_acc - n, d), x.dtype)])

    deg_kernel = functools.partial(
        pl.kernel,
        out_type=jax.ShapeDtypeStruct((NW, n_acc), jnp.float32),
        mesh=mesh,
        scratch_types=[
            pltpu.VMEM((ept,), jnp.int32),
            pltpu.VMEM((n_acc,), jnp.float32),
        ],
        compiler_params=pltpu.CompilerParams(needs_layout_passes=False),
    )(functools.partial(_deg_kernel_body, n_acc, ept))
    hists = deg_kernel(dst_flat)

    bn = NS * NBLK
    grid = n_acc // bn
    g_arr = pl.pallas_call(
        _g_kernel_body,
        grid=(grid,),
        in_specs=[
            pl.BlockSpec((bn, d), lambda i: (i, 0)),
            pl.BlockSpec((d, d), lambda i: (0, 0)),
            pl.BlockSpec((NW, bn), lambda i: (0, i)),
        ],
        out_specs=pl.BlockSpec((bn, d), lambda i: (i, 0)),
        out_shape=jax.ShapeDtypeStruct((n_acc, d), jnp.float32),
    )(x_pad, W, hists)

    edge_kernel = functools.partial(
        pl.kernel,
        out_type=jax.ShapeDtypeStruct((NC, n_acc, d), jnp.float32),
        mesh=mesh,
        scratch_types=[
            pltpu.VMEM((2, SC_CH, CHUNK), jnp.int32),
            pltpu.VMEM((2, SC_CH, CHUNK), jnp.int32),
            pltpu.VMEM((2, CHUNK, d), jnp.float32),
            pltpu.VMEM_SHARED((n_acc, d), jnp.float32),
            pltpu.SemaphoreType.DMA,
            pltpu.SemaphoreType.DMA,
            pltpu.SemaphoreType.DMA,
        ],
        compiler_params=pltpu.CompilerParams(needs_layout_passes=False),
    )(functools.partial(_edge_kernel_body, n_acc, c_t))
    parts = edge_kernel(src_r, dst_r, g_arr)

    out = pl.pallas_call(
        _out_kernel_body,
        grid=(grid,),
        in_specs=[
            pl.BlockSpec((NC, bn, d), lambda i: (0, i, 0)),
            pl.BlockSpec((bn, d), lambda i: (i, 0)),
            pl.BlockSpec((NW, bn), lambda i: (0, i)),
            pl.BlockSpec((1, d), lambda i: (0, 0)),
        ],
        out_specs=pl.BlockSpec((bn, d), lambda i: (i, 0)),
        out_shape=jax.ShapeDtypeStruct((n_acc, d), jnp.float32),
    )(parts, g_arr, hists, b.reshape(1, d))
    return out[:n]


# speedup vs baseline: 25.0237x; 1.8787x over previous
import functools

import jax
import jax.numpy as jnp
from jax import lax
from jax.experimental import pallas as pl
from jax.experimental.pallas import tpu as pltpu
from jax.experimental.pallas import tpu_sc as plsc

NC = 2
NS = 16
NW = NC * NS
LANES = 16

CHUNK = 128
RING = 4
NBLK = 128
FRAC0 = 0.81


def _sc_mesh():
    return plsc.VectorSubcoreMesh(core_axis_name="c", subcore_axis_name="s")


def _deg_kernel_body(n_acc, ept, dst_hbm, hists_hbm, dstv, hist):
    c = lax.axis_index("c")
    s = lax.axis_index("s")
    wid = s * NC + c
    pltpu.sync_copy(dst_hbm.at[wid], dstv)

    zeros16 = jnp.zeros((LANES,), jnp.float32)

    def zero_body(i, carry):
        hist[pl.ds(i * LANES, LANES)] = zeros16
        return carry

    lax.fori_loop(0, n_acc // LANES, zero_body, 0)

    ones16 = jnp.ones((LANES,), jnp.float32)

    def body(i, carry):
        idx = dstv[pl.ds(i * LANES, LANES)]
        plsc.addupdate_scatter(hist, [idx], ones16)
        return carry

    lax.fori_loop(0, ept // LANES, body, 0)
    pltpu.sync_copy(hist, hists_hbm.at[wid])


def _g_kernel_body(x_ref, w_ref, hists_ref, g_ref):
    deg = jnp.sum(hists_ref[...], axis=0) + 1.0
    dinv = lax.rsqrt(deg)
    h = jnp.dot(x_ref[...], w_ref[...], preferred_element_type=jnp.float32)
    g_ref[...] = h * dinv[:, None]


def _edge_kernel_body(n_acc, k0, k1, src_hbm, dst_hbm, g_hbm, part_hbm,
                      srcv, dstv, rows, acc, sg0, sg1, si):
    cid = lax.axis_index("c")
    s = lax.axis_index("s")
    d = g_hbm.shape[1]
    nloc = jnp.where(cid == 0, k0, k1)
    base = jnp.where(cid == 0, s * k0, NS * k0 + s * k1)

    zeros16 = jnp.zeros((LANES,), jnp.float32)
    per_row = d // LANES

    def zb(i, carry):
        r = i // per_row
        j = i % per_row
        rows[0, r, pl.ds(j * LANES, LANES)] = zeros16
        return carry

    lax.fori_loop(0, CHUNK * per_row, zb, 0)

    r_t = n_acc // NS

    def zacc(j, carry):
        pltpu.sync_copy(rows.at[0], acc.at[pl.ds(s * r_t + j * CHUNK, CHUNK)])
        return carry

    lax.fori_loop(0, r_t // CHUNK, zacc, 0)
    plsc.subcore_barrier()

    def stage(j):
        pltpu.async_copy(src_hbm.at[base + j], srcv.at[j % RING], si)
        pltpu.async_copy(dst_hbm.at[base + j], dstv.at[j % RING], si)

    def stage_wait():
        pltpu.make_async_copy(src_hbm.at[base], srcv.at[0], si).wait()
        pltpu.make_async_copy(dst_hbm.at[base], dstv.at[0], si).wait()

    sems = (sg0, sg1)
    for j in range(RING):
        stage(j)
    for b in range(2):
        stage_wait()
        pltpu.async_copy(g_hbm.at[srcv.at[b]], rows.at[b], sems[b])

    def body(j2, carry):
        for b in range(2):
            j = j2 * 2 + b
            pltpu.make_async_copy(g_hbm.at[srcv.at[j % RING]], rows.at[b],
                                  sems[b]).wait()
            pltpu.sync_copy(rows.at[b], acc.at[dstv.at[j % RING]], add=True)

            @pl.when(j + RING < nloc)
            def _():
                stage(j + RING)

            @pl.when(j + 2 < nloc)
            def _():
                stage_wait()
                pltpu.async_copy(g_hbm.at[srcv.at[(j + 2) % RING]],
                                 rows.at[b], sems[b])
        return carry

    lax.fori_loop(0, nloc // 2, body, 0)
    plsc.subcore_barrier()

    pltpu.sync_copy(acc.at[pl.ds(s * r_t, r_t)],
                    part_hbm.at[cid, pl.ds(s * r_t, r_t)])


def _out_kernel_body(p_ref, g_ref, hists_ref, b_ref, o_ref):
    deg = jnp.sum(hists_ref[...], axis=0) + 1.0
    dinv = lax.rsqrt(deg)
    tot = p_ref[0] + p_ref[1] + g_ref[...]
    o_ref[...] = tot * dinv[:, None] + b_ref[...]


def kernel(x, edge_index, W, b):
    n, d = x.shape
    e = edge_index.shape[1]

    kt = -(-e // (NS * CHUNK))
    if kt % 2:
        kt += 1
    k0 = max(RING, min(kt - RING, 2 * round(kt * FRAC0 / 2)))
    k1 = kt - k0
    t_chunks = NS * kt
    e_pad = t_chunks * CHUNK

    n_acc = -(-(n + 1) // (NS * NBLK)) * (NS * NBLK)

    src = edge_index[0].astype(jnp.int32)
    dst = edge_index[1].astype(jnp.int32)
    pad = e_pad - e
    src_p = jnp.concatenate([src, jnp.zeros((pad,), jnp.int32)])
    dst_p = jnp.concatenate([dst, jnp.full((pad,), n, jnp.int32)])
    src_r = src_p.reshape(t_chunks, CHUNK)
    dst_r = dst_p.reshape(t_chunks, CHUNK)
    ept = e_pad // NW
    dst_flat = dst_p.reshape(NW, ept)

    mesh = _sc_mesh()

    x_pad = jnp.concatenate([x, jnp.zeros((n_acc - n, d), x.dtype)])

    deg_kernel = functools.partial(
        pl.kernel,
        out_type=jax.ShapeDtypeStruct((NW, n_acc), jnp.float32),
        mesh=mesh,
        scratch_types=[
            pltpu.VMEM((ept,), jnp.int32),
            pltpu.VMEM((n_acc,), jnp.float32),
        ],
        compiler_params=pltpu.CompilerParams(needs_layout_passes=False),
    )(functools.partial(_deg_kernel_body, n_acc, ept))
    hists = deg_kernel(dst_flat)

    bn = NS * NBLK
    grid = n_acc // bn
    g_arr = pl.pallas_call(
        _g_kernel_body,
        grid=(grid,),
        in_specs=[
            pl.BlockSpec((bn, d), lambda i: (i, 0)),
            pl.BlockSpec((d, d), lambda i: (0, 0)),
            pl.BlockSpec((NW, bn), lambda i: (0, i)),
        ],
        out_specs=pl.BlockSpec((bn, d), lambda i: (i, 0)),
        out_shape=jax.ShapeDtypeStruct((n_acc, d), jnp.float32),
    )(x_pad, W, hists)

    edge_kernel = functools.partial(
        pl.kernel,
        out_type=jax.ShapeDtypeStruct((NC, n_acc, d), jnp.float32),
        mesh=mesh,
        scratch_types=[
            pltpu.VMEM((RING, CHUNK), jnp.int32),
            pltpu.VMEM((RING, CHUNK), jnp.int32),
            pltpu.VMEM((2, CHUNK, d), jnp.float32),
            pltpu.VMEM_SHARED((n_acc, d), jnp.float32),
            pltpu.SemaphoreType.DMA,
            pltpu.SemaphoreType.DMA,
            pltpu.SemaphoreType.DMA,
        ],
        compiler_params=pltpu.CompilerParams(needs_layout_passes=False),
    )(functools.partial(_edge_kernel_body, n_acc, k0, k1))
    parts = edge_kernel(src_r, dst_r, g_arr)

    out = pl.pallas_call(
        _out_kernel_body,
        grid=(grid,),
        in_specs=[
            pl.BlockSpec((NC, bn, d), lambda i: (0, i, 0)),
            pl.BlockSpec((bn, d), lambda i: (i, 0)),
            pl.BlockSpec((NW, bn), lambda i: (0, i)),
            pl.BlockSpec((1, d), lambda i: (0, 0)),
        ],
        out_specs=pl.BlockSpec((bn, d), lambda i: (i, 0)),
        out_shape=jax.ShapeDtypeStruct((n_acc, d), jnp.float32),
    )(parts, g_arr, hists, b.reshape(1, d))
    return out[:n]


# speedup vs baseline: 25.5880x; 1.0226x over previous
import functools

import jax
import jax.numpy as jnp
from jax import lax
from jax.experimental import pallas as pl
from jax.experimental.pallas import tpu as pltpu
from jax.experimental.pallas import tpu_sc as plsc

NC = 2
NS = 16
NW = NC * NS
LANES = 16

CHUNK = 128
RING = 4
NBLK = 128
FRAC0 = 0.873


def _sc_mesh():
    return plsc.VectorSubcoreMesh(core_axis_name="c", subcore_axis_name="s")


def _deg_kernel_body(n_acc, ept, dst_hbm, hists_hbm, dstv, hist):
    c = lax.axis_index("c")
    s = lax.axis_index("s")
    wid = s * NC + c
    pltpu.sync_copy(dst_hbm.at[wid], dstv)

    zeros16 = jnp.zeros((LANES,), jnp.float32)

    def zero_body(i, carry):
        hist[pl.ds(i * LANES, LANES)] = zeros16
        return carry

    lax.fori_loop(0, n_acc // LANES, zero_body, 0)

    ones16 = jnp.ones((LANES,), jnp.float32)

    def body(i, carry):
        idx = dstv[pl.ds(i * LANES, LANES)]
        plsc.addupdate_scatter(hist, [idx], ones16)
        return carry

    lax.fori_loop(0, ept // LANES, body, 0)
    pltpu.sync_copy(hist, hists_hbm.at[wid])


def _g_kernel_body(x_ref, w_ref, hists_ref, g_ref):
    deg = jnp.sum(hists_ref[...], axis=0) + 1.0
    dinv = lax.rsqrt(deg)
    h = jnp.dot(x_ref[...], w_ref[...], preferred_element_type=jnp.float32)
    g_ref[...] = h * dinv[:, None]


def _edge_kernel_body(n_acc, k0, k1, src_hbm, dst_hbm, g_hbm, part_hbm,
                      srcv, dstv, rows, acc, sg0, sg1, si):
    cid = lax.axis_index("c")
    s = lax.axis_index("s")
    d = g_hbm.shape[1]
    nloc = jnp.where(cid == 0, k0, k1)
    base = jnp.where(cid == 0, s * k0, NS * k0 + s * k1)

    zeros16 = jnp.zeros((LANES,), jnp.float32)
    per_row = d // LANES

    def zb(i, carry):
        r = i // per_row
        j = i % per_row
        rows[0, r, pl.ds(j * LANES, LANES)] = zeros16
        return carry

    lax.fori_loop(0, CHUNK * per_row, zb, 0)

    r_t = n_acc // NS

    def zacc(j, carry):
        pltpu.sync_copy(rows.at[0], acc.at[pl.ds(s * r_t + j * CHUNK, CHUNK)])
        return carry

    lax.fori_loop(0, r_t // CHUNK, zacc, 0)
    plsc.subcore_barrier()

    def stage(j):
        pltpu.async_copy(src_hbm.at[base + j], srcv.at[j % RING], si)
        pltpu.async_copy(dst_hbm.at[base + j], dstv.at[j % RING], si)

    def stage_wait():
        pltpu.make_async_copy(src_hbm.at[base], srcv.at[0], si).wait()
        pltpu.make_async_copy(dst_hbm.at[base], dstv.at[0], si).wait()

    sems = (sg0, sg1)
    for j in range(RING):
        stage(j)
    for b in range(2):
        stage_wait()
        pltpu.async_copy(g_hbm.at[srcv.at[b]], rows.at[b], sems[b])

    def body(j2, carry):
        for b in range(2):
            j = j2 * 2 + b
            pltpu.make_async_copy(g_hbm.at[srcv.at[j % RING]], rows.at[b],
                                  sems[b]).wait()
            pltpu.sync_copy(rows.at[b], acc.at[dstv.at[j % RING]], add=True)

            @pl.when(j + RING < nloc)
            def _():
                stage(j + RING)

            @pl.when(j + 2 < nloc)
            def _():
                stage_wait()
                pltpu.async_copy(g_hbm.at[srcv.at[(j + 2) % RING]],
                                 rows.at[b], sems[b])
        return carry

    lax.fori_loop(0, nloc // 2, body, 0)
    plsc.subcore_barrier()

    pltpu.sync_copy(acc.at[pl.ds(s * r_t, r_t)],
                    part_hbm.at[cid, pl.ds(s * r_t, r_t)])


def _out_kernel_body(p_ref, g_ref, hists_ref, b_ref, o_ref):
    deg = jnp.sum(hists_ref[...], axis=0) + 1.0
    dinv = lax.rsqrt(deg)
    tot = p_ref[0] + p_ref[1] + g_ref[...]
    o_ref[...] = tot * dinv[:, None] + b_ref[...]


def kernel(x, edge_index, W, b):
    n, d = x.shape
    e = edge_index.shape[1]

    kt = -(-e // (NS * CHUNK))
    if kt % 2:
        kt += 1
    k0 = max(RING, min(kt - RING, 2 * round(kt * FRAC0 / 2)))
    k1 = kt - k0
    t_chunks = NS * kt
    e_pad = t_chunks * CHUNK

    n_acc = -(-(n + 1) // (NS * NBLK)) * (NS * NBLK)

    src = edge_index[0].astype(jnp.int32)
    dst = edge_index[1].astype(jnp.int32)
    pad = e_pad - e
    src_p = jnp.concatenate([src, jnp.zeros((pad,), jnp.int32)])
    dst_p = jnp.concatenate([dst, jnp.full((pad,), n, jnp.int32)])
    src_r = src_p.reshape(t_chunks, CHUNK)
    dst_r = dst_p.reshape(t_chunks, CHUNK)
    ept = e_pad // NW
    dst_flat = dst_p.reshape(NW, ept)

    mesh = _sc_mesh()

    x_pad = jnp.concatenate([x, jnp.zeros((n_acc - n, d), x.dtype)])

    deg_kernel = functools.partial(
        pl.kernel,
        out_type=jax.ShapeDtypeStruct((NW, n_acc), jnp.float32),
        mesh=mesh,
        scratch_types=[
            pltpu.VMEM((ept,), jnp.int32),
            pltpu.VMEM((n_acc,), jnp.float32),
        ],
        compiler_params=pltpu.CompilerParams(needs_layout_passes=False),
    )(functools.partial(_deg_kernel_body, n_acc, ept))
    hists = deg_kernel(dst_flat)

    bn = NS * NBLK
    grid = n_acc // bn
    g_arr = pl.pallas_call(
        _g_kernel_body,
        grid=(grid,),
        in_specs=[
            pl.BlockSpec((bn, d), lambda i: (i, 0)),
            pl.BlockSpec((d, d), lambda i: (0, 0)),
            pl.BlockSpec((NW, bn), lambda i: (0, i)),
        ],
        out_specs=pl.BlockSpec((bn, d), lambda i: (i, 0)),
        out_shape=jax.ShapeDtypeStruct((n_acc, d), jnp.float32),
    )(x_pad, W, hists)

    edge_kernel = functools.partial(
        pl.kernel,
        out_type=jax.ShapeDtypeStruct((NC, n_acc, d), jnp.float32),
        mesh=mesh,
        scratch_types=[
            pltpu.VMEM((RING, CHUNK), jnp.int32),
            pltpu.VMEM((RING, CHUNK), jnp.int32),
            pltpu.VMEM((2, CHUNK, d), jnp.float32),
            pltpu.VMEM_SHARED((n_acc, d), jnp.float32),
            pltpu.SemaphoreType.DMA,
            pltpu.SemaphoreType.DMA,
            pltpu.SemaphoreType.DMA,
        ],
        compiler_params=pltpu.CompilerParams(needs_layout_passes=False),
    )(functools.partial(_edge_kernel_body, n_acc, k0, k1))
    parts = edge_kernel(src_r, dst_r, g_arr)

    out = pl.pallas_call(
        _out_kernel_body,
        grid=(grid,),
        in_specs=[
            pl.BlockSpec((NC, bn, d), lambda i: (0, i, 0)),
            pl.BlockSpec((bn, d), lambda i: (i, 0)),
            pl.BlockSpec((NW, bn), lambda i: (0, i)),
            pl.BlockSpec((1, d), lambda i: (0, 0)),
        ],
        out_specs=pl.BlockSpec((bn, d), lambda i: (i, 0)),
        out_shape=jax.ShapeDtypeStruct((n_acc, d), jnp.float32),
    )(parts, g_arr, hists, b.reshape(1, d))
    return out[:n]
